# baseline (speedup 1.0000x reference)
import jax
import jax.numpy as jnp
import numpy as np
from jax.experimental import pallas as pl
from jax.experimental.pallas import tpu as pltpu

_SCALES = (128, 64, 32)
_HIDDEN = 64
_ITERS = 2
_BJ = 8

_F32 = jnp.float32


def _dot(a, b):
    return jnp.dot(a, b, preferred_element_type=_F32)


def _dot_nt(a, b):
    return jax.lax.dot_general(a, b, (((1,), (1,)), ((), ())),
                               preferred_element_type=_F32)


def _dot_tn(a, b):
    return jax.lax.dot_general(a, b, (((0,), (0,)), ((), ())),
                               preferred_element_type=_F32)


def _mpnn_body(jets_ref, wemb_ref, bemb_ref,
               wa0_ref, wm0_ref, wuh0_ref, wum0_ref, bu0_ref,
               wa1_ref, wm1_ref, wuh1_ref, wum1_ref, bu1_ref,
               wa2_ref, wm2_ref, wuh2_ref, wum2_ref, bu2_ref,
               wp0_ref, wp1_ref, wread_ref, bread_ref, out_ref):
    scale = np.float32(1.0 / np.sqrt(_HIDDEN))
    h = jnp.tanh(_dot(jets_ref[...], wemb_ref[...]) + bemb_ref[...])

    was = (wa0_ref, wa1_ref, wa2_ref)
    wms = (wm0_ref, wm1_ref, wm2_ref)
    wuhs = (wuh0_ref, wuh1_ref, wuh2_ref)
    wums = (wum0_ref, wum1_ref, wum2_ref)
    bus = (bu0_ref, bu1_ref, bu2_ref)
    wps = (wp0_ref, wp1_ref)

    for i, n in enumerate(_SCALES):
        for _ in range(_ITERS):
            h_adj = _dot(h, was[i][...])
            h_msg = _dot(h, wms[i][...])
            h_upd = _dot(h, wuhs[i][...])
            m_parts = []
            for j in range(_BJ):
                hj = h[j * n:(j + 1) * n, :]
                logits = _dot_nt(h_adj[j * n:(j + 1) * n, :], hj) * scale
                logits = logits - jnp.max(logits, axis=-1, keepdims=True)
                e = jnp.exp(logits)
                attn = e / jnp.sum(e, axis=-1, keepdims=True)
                m_parts.append(_dot(attn, h_msg[j * n:(j + 1) * n, :]))
            m = jnp.concatenate(m_parts, axis=0)
            h = jnp.tanh(h_upd + _dot(m, wums[i][...]) + bus[i][...])
        if i < len(_SCALES) - 1:
            p = _dot(h, wps[i][...])
            parts = []
            for j in range(_BJ):
                pj = p[j * n:(j + 1) * n, :]
                pj = pj - jnp.max(pj, axis=0, keepdims=True)
                e = jnp.exp(pj)
                attn = e / jnp.sum(e, axis=0, keepdims=True)
                parts.append(_dot_tn(attn, h[j * n:(j + 1) * n, :]))
            h = jnp.concatenate(parts, axis=0)

    n_last = _SCALES[-1]
    sums = [jnp.sum(h[j * n_last:(j + 1) * n_last, :], axis=0, keepdims=True)
            for j in range(_BJ)]
    s = jnp.concatenate(sums, axis=0)
    out_ref[...] = jnp.tanh(_dot(s, wread_ref[...]) + bread_ref[...])


def kernel(jets, W_emb, b_emb, W_adj0, W_msg0, W_upd0, b_upd0,
           W_adj1, W_msg1, W_upd1, b_upd1,
           W_adj2, W_msg2, W_upd2, b_upd2,
           W_pool0, W_pool1, W_read, b_read):
    b, n0, d_in = jets.shape
    jets2 = jets.reshape(b * n0, d_in)
    bemb = b_emb.reshape(1, _HIDDEN)
    bread = b_read.reshape(1, _HIDDEN)
    wuh0, wum0 = W_upd0[:_HIDDEN], W_upd0[_HIDDEN:]
    wuh1, wum1 = W_upd1[:_HIDDEN], W_upd1[_HIDDEN:]
    wuh2, wum2 = W_upd2[:_HIDDEN], W_upd2[_HIDDEN:]
    bu0 = b_upd0.reshape(1, _HIDDEN)
    bu1 = b_upd1.reshape(1, _HIDDEN)
    bu2 = b_upd2.reshape(1, _HIDDEN)

    def full(arr):
        return pl.BlockSpec(arr.shape, lambda i: (0,) * arr.ndim)

    operands = (jets2, W_emb, bemb,
                W_adj0, W_msg0, wuh0, wum0, bu0,
                W_adj1, W_msg1, wuh1, wum1, bu1,
                W_adj2, W_msg2, wuh2, wum2, bu2,
                W_pool0, W_pool1, W_read, bread)
    in_specs = [pl.BlockSpec((_BJ * n0, d_in), lambda i: (i, 0))]
    in_specs += [full(a) for a in operands[1:]]

    out = pl.pallas_call(
        _mpnn_body,
        grid=(b // _BJ,),
        in_specs=in_specs,
        out_specs=pl.BlockSpec((_BJ, _HIDDEN), lambda i: (i, 0)),
        out_shape=jax.ShapeDtypeStruct((b, _HIDDEN), _F32),
        compiler_params=pltpu.CompilerParams(
            dimension_semantics=("arbitrary",)),
    )(*operands)
    return out


# speedup vs baseline: 1.2467x; 1.2467x over previous
import jax
import jax.numpy as jnp
import numpy as np
from jax.experimental import pallas as pl
from jax.experimental.pallas import tpu as pltpu

_SCALES = (128, 64, 32)
_HIDDEN = 64
_ITERS = 2
_BJ = 8

_F32 = jnp.float32
_BF16 = jnp.bfloat16


def _dot_bf(a, b):
    return jnp.dot(a.astype(_BF16), b.astype(_BF16),
                   preferred_element_type=_F32)


def _dot_f32(a, b):
    return jnp.dot(a, b, preferred_element_type=_F32)


def _dot_nt_bf(a, b):
    return jax.lax.dot_general(a.astype(_BF16), b.astype(_BF16),
                               (((1,), (1,)), ((), ())),
                               preferred_element_type=_F32)


def _dot_tn_bf(a, b):
    return jax.lax.dot_general(a.astype(_BF16), b.astype(_BF16),
                               (((0,), (0,)), ((), ())),
                               preferred_element_type=_F32)


def _mpnn_body(jets_ref, wemb_ref, bemb_ref,
               wa0_ref, wm0_ref, wuh0_ref, wum0_ref, bu0_ref,
               wa1_ref, wm1_ref, wuh1_ref, wum1_ref, bu1_ref,
               wa2_ref, wm2_ref, wuh2_ref, wum2_ref, bu2_ref,
               wp0_ref, wp1_ref, wread_ref, bread_ref, out_ref):
    scale = np.float32(1.0 / np.sqrt(_HIDDEN))
    h = jnp.tanh(_dot_f32(jets_ref[...], wemb_ref[...]) + bemb_ref[...])

    was = (wa0_ref, wa1_ref, wa2_ref)
    wms = (wm0_ref, wm1_ref, wm2_ref)
    wuhs = (wuh0_ref, wuh1_ref, wuh2_ref)
    wums = (wum0_ref, wum1_ref, wum2_ref)
    bus = (bu0_ref, bu1_ref, bu2_ref)
    wps = (wp0_ref, wp1_ref)

    for i, n in enumerate(_SCALES):
        for _ in range(_ITERS):
            hb = h.astype(_BF16)
            h_adj = _dot_bf(hb, was[i][...])
            h_msg = _dot_bf(hb, wms[i][...]).astype(_BF16)
            h_upd = _dot_f32(h, wuhs[i][...])
            m_parts = []
            for j in range(_BJ):
                logits = _dot_nt_bf(h_adj[j * n:(j + 1) * n, :],
                                    hb[j * n:(j + 1) * n, :]) * scale
                logits = logits - jnp.max(logits, axis=-1, keepdims=True)
                e = jnp.exp(logits)
                r = 1.0 / jnp.sum(e, axis=-1, keepdims=True)
                m_parts.append(
                    _dot_bf(e, h_msg[j * n:(j + 1) * n, :]) * r)
            m = jnp.concatenate(m_parts, axis=0)
            h = jnp.tanh(h_upd + _dot_f32(m, wums[i][...]) + bus[i][...])
        if i < len(_SCALES) - 1:
            p = _dot_bf(h, wps[i][...])
            parts = []
            for j in range(_BJ):
                pj = p[j * n:(j + 1) * n, :]
                pj = pj - jnp.max(pj, axis=0, keepdims=True)
                e = jnp.exp(pj)
                attn = e * (1.0 / jnp.sum(e, axis=0, keepdims=True))
                parts.append(_dot_tn_bf(attn, h[j * n:(j + 1) * n, :]))
            h = jnp.concatenate(parts, axis=0)

    n_last = _SCALES[-1]
    sums = [jnp.sum(h[j * n_last:(j + 1) * n_last, :], axis=0, keepdims=True)
            for j in range(_BJ)]
    s = jnp.concatenate(sums, axis=0)
    out_ref[...] = jnp.tanh(_dot_f32(s, wread_ref[...]) + bread_ref[...])


def kernel(jets, W_emb, b_emb, W_adj0, W_msg0, W_upd0, b_upd0,
           W_adj1, W_msg1, W_upd1, b_upd1,
           W_adj2, W_msg2, W_upd2, b_upd2,
           W_pool0, W_pool1, W_read, b_read):
    b, n0, d_in = jets.shape
    jets2 = jets.reshape(b * n0, d_in)
    bemb = b_emb.reshape(1, _HIDDEN)
    bread = b_read.reshape(1, _HIDDEN)
    wuh0, wum0 = W_upd0[:_HIDDEN], W_upd0[_HIDDEN:]
    wuh1, wum1 = W_upd1[:_HIDDEN], W_upd1[_HIDDEN:]
    wuh2, wum2 = W_upd2[:_HIDDEN], W_upd2[_HIDDEN:]
    bu0 = b_upd0.reshape(1, _HIDDEN)
    bu1 = b_upd1.reshape(1, _HIDDEN)
    bu2 = b_upd2.reshape(1, _HIDDEN)
    wa0, wa1, wa2 = (w.astype(_BF16) for w in (W_adj0, W_adj1, W_adj2))
    wm0, wm1, wm2 = (w.astype(_BF16) for w in (W_msg0, W_msg1, W_msg2))
    wp0, wp1 = W_pool0.astype(_BF16), W_pool1.astype(_BF16)

    def full(arr):
        return pl.BlockSpec(arr.shape, lambda i: (0,) * arr.ndim)

    operands = (jets2, W_emb, bemb,
                wa0, wm0, wuh0, wum0, bu0,
                wa1, wm1, wuh1, wum1, bu1,
                wa2, wm2, wuh2, wum2, bu2,
                wp0, wp1, W_read, bread)
    in_specs = [pl.BlockSpec((_BJ * n0, d_in), lambda i: (i, 0))]
    in_specs += [full(a) for a in operands[1:]]

    out = pl.pallas_call(
        _mpnn_body,
        grid=(b // _BJ,),
        in_specs=in_specs,
        out_specs=pl.BlockSpec((_BJ, _HIDDEN), lambda i: (i, 0)),
        out_shape=jax.ShapeDtypeStruct((b, _HIDDEN), _F32),
        compiler_params=pltpu.CompilerParams(
            dimension_semantics=("parallel",)),
    )(*operands)
    return out


# speedup vs baseline: 3.9761x; 3.1892x over previous
import jax
import jax.numpy as jnp
import numpy as np
from jax.experimental import pallas as pl
from jax.experimental.pallas import tpu as pltpu

_SCALES = (128, 64, 32)
_HIDDEN = 64
_ITERS = 2
_BJ = 16
_H1 = _HIDDEN + 1

_F32 = jnp.float32
_BF16 = jnp.bfloat16


def _dot_bf(a, b):
    return jnp.dot(a.astype(_BF16), b.astype(_BF16),
                   preferred_element_type=_F32)


def _dot_f32(a, b):
    return jnp.dot(a, b, preferred_element_type=_F32)


def _dot_nt_bf(a, b):
    return jax.lax.dot_general(a.astype(_BF16), b.astype(_BF16),
                               (((1,), (1,)), ((), ())),
                               preferred_element_type=_F32)


def _dot_tn_bf(a, b):
    return jax.lax.dot_general(a.astype(_BF16), b.astype(_BF16),
                               (((0,), (0,)), ((), ())),
                               preferred_element_type=_F32)


def _mpnn_body(jets_ref, wemb_ref, bemb_ref,
               wa0_ref, wm0_ref, wuh0_ref, wum0_ref, bu0_ref,
               wa1_ref, wm1_ref, wuh1_ref, wum1_ref, bu1_ref,
               wa2_ref, wm2_ref, wuh2_ref, wum2_ref, bu2_ref,
               wp0_ref, wp1_ref, wread_ref, bread_ref, out_ref):
    h = jnp.tanh(_dot_f32(jets_ref[...], wemb_ref[...]) + bemb_ref[...])

    was = (wa0_ref, wa1_ref, wa2_ref)
    wms = (wm0_ref, wm1_ref, wm2_ref)
    wuhs = (wuh0_ref, wuh1_ref, wuh2_ref)
    wums = (wum0_ref, wum1_ref, wum2_ref)
    bus = (bu0_ref, bu1_ref, bu2_ref)
    wps = (wp0_ref, wp1_ref)

    for i, n in enumerate(_SCALES):
        for _ in range(_ITERS):
            hb = h.astype(_BF16)
            h_adj = _dot_bf(hb, was[i][...])
            h_msg = _dot_bf(hb, wms[i][...]).astype(_BF16)
            h_upd = _dot_f32(h, wuhs[i][...])
            lgs = [_dot_nt_bf(h_adj[j * n:(j + 1) * n, :],
                              hb[j * n:(j + 1) * n, :]) for j in range(_BJ)]
            es = [jnp.exp(lg) for lg in lgs]
            raws = [_dot_bf(es[j], h_msg[j * n:(j + 1) * n, :])
                    for j in range(_BJ)]
            m = jnp.concatenate(
                [raw * (1.0 / raw[:, _HIDDEN:_H1]) for raw in raws], axis=0)
            h = jnp.tanh(h_upd + _dot_f32(m, wums[i][...]) + bus[i][...])
        if i < len(_SCALES) - 1:
            p = _dot_bf(h, wps[i][...])
            e = jnp.exp(p)
            raws = [_dot_tn_bf(e[j * n:(j + 1) * n, :],
                               h[j * n:(j + 1) * n, :])
                    for j in range(_BJ)]
            h = jnp.concatenate(
                [raw * (1.0 / raw[:, _HIDDEN:_H1]) for raw in raws], axis=0)

    n_last = _SCALES[-1]
    rows = _BJ * n_last
    sel = (jax.lax.broadcasted_iota(jnp.int32, (_BJ, rows), 0)
           == jax.lax.broadcasted_iota(jnp.int32, (_BJ, rows), 1) // n_last)
    s = _dot_f32(sel.astype(_F32), h)
    out_ref[...] = jnp.tanh(_dot_f32(s, wread_ref[...]) + bread_ref[...])


def kernel(jets, W_emb, b_emb, W_adj0, W_msg0, W_upd0, b_upd0,
           W_adj1, W_msg1, W_upd1, b_upd1,
           W_adj2, W_msg2, W_upd2, b_upd2,
           W_pool0, W_pool1, W_read, b_read):
    b, n0, d_in = jets.shape
    h = _HIDDEN
    jets2 = jets.reshape(b * n0, d_in)
    wemb = jnp.pad(W_emb, ((0, 0), (0, 1)))
    bemb = jnp.concatenate([b_emb, jnp.full((1,), 30.0, _F32)]).reshape(1, _H1)
    bread = b_read.reshape(1, h)

    scale = np.float32(1.0 / np.sqrt(h))

    def pad_rc(w):
        return jnp.pad(w, ((0, 1), (0, 1)))

    wa0, wa1, wa2 = (pad_rc(w * scale).astype(_BF16)
                     for w in (W_adj0, W_adj1, W_adj2))
    wm0, wm1, wm2 = (pad_rc(w).at[h, h].set(1.0).astype(_BF16)
                     for w in (W_msg0, W_msg1, W_msg2))
    wuh0, wuh1, wuh2 = (pad_rc(w[:h]) for w in (W_upd0, W_upd1, W_upd2))
    wum0, wum1, wum2 = (pad_rc(w[h:]) for w in (W_upd0, W_upd1, W_upd2))
    bu0, bu1, bu2 = (
        jnp.concatenate([x, jnp.full((1,), 30.0, _F32)]).reshape(1, _H1)
        for x in (b_upd0, b_upd1, b_upd2))
    wp0, wp1 = (jnp.pad(w, ((0, 1), (0, 0))).astype(_BF16)
                for w in (W_pool0, W_pool1))
    wread = jnp.pad(W_read, ((0, 1), (0, 0)))

    def full(arr):
        return pl.BlockSpec(arr.shape, lambda i: (0,) * arr.ndim)

    operands = (jets2, wemb, bemb,
                wa0, wm0, wuh0, wum0, bu0,
                wa1, wm1, wuh1, wum1, bu1,
                wa2, wm2, wuh2, wum2, bu2,
                wp0, wp1, wread, bread)
    in_specs = [pl.BlockSpec((_BJ * n0, d_in), lambda i: (i, 0))]
    in_specs += [full(a) for a in operands[1:]]

    out = pl.pallas_call(
        _mpnn_body,
        grid=(b // _BJ,),
        in_specs=in_specs,
        out_specs=pl.BlockSpec((_BJ, h), lambda i: (i, 0)),
        out_shape=jax.ShapeDtypeStruct((b, h), _F32),
        compiler_params=pltpu.CompilerParams(
            dimension_semantics=("parallel",)),
    )(*operands)
    return out


# speedup vs baseline: 4.8804x; 1.2274x over previous
import jax
import jax.numpy as jnp
import numpy as np
from jax.experimental import pallas as pl
from jax.experimental.pallas import tpu as pltpu

_SCALES = (128, 64, 32)
_HIDDEN = 64
_ITERS = 2
_BJ = 32
_H1 = _HIDDEN + 1

_F32 = jnp.float32
_BF16 = jnp.bfloat16


def _dot_bf(a, b):
    return jnp.dot(a.astype(_BF16), b.astype(_BF16),
                   preferred_element_type=_F32)


def _dot_f32(a, b):
    return jnp.dot(a, b, preferred_element_type=_F32)


def _dot_nt_bf(a, b):
    return jax.lax.dot_general(a.astype(_BF16), b.astype(_BF16),
                               (((1,), (1,)), ((), ())),
                               preferred_element_type=_F32)


def _dot_tn_bf(a, b):
    return jax.lax.dot_general(a.astype(_BF16), b.astype(_BF16),
                               (((0,), (0,)), ((), ())),
                               preferred_element_type=_F32)


def _mpnn_body(jets_ref, wemb_ref, bemb_ref,
               wa0_ref, wm0_ref, wuh0_ref, wum0_ref, bu0_ref,
               wa1_ref, wm1_ref, wuh1_ref, wum1_ref, bu1_ref,
               wa2_ref, wm2_ref, wuh2_ref, wum2_ref, bu2_ref,
               wp0_ref, wp1_ref, wread_ref, bread_ref, out_ref):
    h = jnp.tanh(_dot_f32(jets_ref[...], wemb_ref[...]) + bemb_ref[...])

    was = (wa0_ref, wa1_ref, wa2_ref)
    wms = (wm0_ref, wm1_ref, wm2_ref)
    wuhs = (wuh0_ref, wuh1_ref, wuh2_ref)
    wums = (wum0_ref, wum1_ref, wum2_ref)
    bus = (bu0_ref, bu1_ref, bu2_ref)
    wps = (wp0_ref, wp1_ref)

    for i, n in enumerate(_SCALES):
        for _ in range(_ITERS):
            hb = h.astype(_BF16)
            h_adj = _dot_bf(hb, was[i][...])
            h_msg = _dot_bf(hb, wms[i][...]).astype(_BF16)
            h_upd = _dot_f32(h, wuhs[i][...])
            lgs = [_dot_nt_bf(h_adj[j * n:(j + 1) * n, :],
                              hb[j * n:(j + 1) * n, :]) for j in range(_BJ)]
            es = [jnp.exp(lg) for lg in lgs]
            raws = [_dot_bf(es[j], h_msg[j * n:(j + 1) * n, :])
                    for j in range(_BJ)]
            m = jnp.concatenate(
                [raw * (1.0 / raw[:, _HIDDEN:_H1]) for raw in raws], axis=0)
            h = jnp.tanh(h_upd + _dot_f32(m, wums[i][...]) + bus[i][...])
        if i < len(_SCALES) - 1:
            p = _dot_bf(h, wps[i][...])
            e = jnp.exp(p)
            raws = [_dot_tn_bf(e[j * n:(j + 1) * n, :],
                               h[j * n:(j + 1) * n, :])
                    for j in range(_BJ)]
            h = jnp.concatenate(
                [raw * (1.0 / raw[:, _HIDDEN:_H1]) for raw in raws], axis=0)

    n_last = _SCALES[-1]
    rows = _BJ * n_last
    sel = (jax.lax.broadcasted_iota(jnp.int32, (_BJ, rows), 0)
           == jax.lax.broadcasted_iota(jnp.int32, (_BJ, rows), 1) // n_last)
    s = _dot_f32(sel.astype(_F32), h)
    out_ref[...] = jnp.tanh(_dot_f32(s, wread_ref[...]) + bread_ref[...])


def kernel(jets, W_emb, b_emb, W_adj0, W_msg0, W_upd0, b_upd0,
           W_adj1, W_msg1, W_upd1, b_upd1,
           W_adj2, W_msg2, W_upd2, b_upd2,
           W_pool0, W_pool1, W_read, b_read):
    b, n0, d_in = jets.shape
    h = _HIDDEN
    jets2 = jets.reshape(b * n0, d_in)
    wemb = jnp.pad(W_emb, ((0, 0), (0, 1)))
    bemb = jnp.concatenate([b_emb, jnp.full((1,), 30.0, _F32)]).reshape(1, _H1)
    bread = b_read.reshape(1, h)

    scale = np.float32(1.0 / np.sqrt(h))

    def pad_rc(w):
        return jnp.pad(w, ((0, 1), (0, 1)))

    wa0, wa1, wa2 = (pad_rc(w * scale).astype(_BF16)
                     for w in (W_adj0, W_adj1, W_adj2))
    wm0, wm1, wm2 = (pad_rc(w).at[h, h].set(1.0).astype(_BF16)
                     for w in (W_msg0, W_msg1, W_msg2))
    wuh0, wuh1, wuh2 = (pad_rc(w[:h]) for w in (W_upd0, W_upd1, W_upd2))
    wum0, wum1, wum2 = (pad_rc(w[h:]) for w in (W_upd0, W_upd1, W_upd2))
    bu0, bu1, bu2 = (
        jnp.concatenate([x, jnp.full((1,), 30.0, _F32)]).reshape(1, _H1)
        for x in (b_upd0, b_upd1, b_upd2))
    wp0, wp1 = (jnp.pad(w, ((0, 1), (0, 0))).astype(_BF16)
                for w in (W_pool0, W_pool1))
    wread = jnp.pad(W_read, ((0, 1), (0, 0)))

    def full(arr):
        return pl.BlockSpec(arr.shape, lambda i: (0,) * arr.ndim)

    operands = (jets2, wemb, bemb,
                wa0, wm0, wuh0, wum0, bu0,
                wa1, wm1, wuh1, wum1, bu1,
                wa2, wm2, wuh2, wum2, bu2,
                wp0, wp1, wread, bread)
    in_specs = [pl.BlockSpec((_BJ * n0, d_in), lambda i: (i, 0))]
    in_specs += [full(a) for a in operands[1:]]

    out = pl.pallas_call(
        _mpnn_body,
        grid=(b // _BJ,),
        in_specs=in_specs,
        out_specs=pl.BlockSpec((_BJ, h), lambda i: (i, 0)),
        out_shape=jax.ShapeDtypeStruct((b, h), _F32),
        compiler_params=pltpu.CompilerParams(
            dimension_semantics=("parallel",)),
    )(*operands)
    return out
